# baseline (speedup 1.0000x reference)
import functools

import jax
import jax.numpy as jnp
from jax import lax
from jax.experimental import pallas as pl
from jax.experimental.pallas import tpu as pltpu
from jax.experimental.pallas import tpu_sc as plsc

VOCAB = 100000
EMBED = 64
DOUT = 128

NC = 2
NS = 16
NW = NC * NS
LANES = 16


def _project_table(table, W, b2d):
    BLK = 4000

    def body(t_ref, w_ref, b_ref, o_ref):
        o_ref[...] = (
            jnp.dot(t_ref[...], w_ref[...], preferred_element_type=jnp.float32)
            + b_ref[...]
        )

    return pl.pallas_call(
        body,
        grid=(VOCAB // BLK,),
        in_specs=[
            pl.BlockSpec((BLK, EMBED), lambda i: (i, 0)),
            pl.BlockSpec((EMBED, DOUT), lambda i: (0, 0)),
            pl.BlockSpec((1, DOUT), lambda i: (0, 0)),
        ],
        out_specs=pl.BlockSpec((BLK, DOUT), lambda i: (i, 0)),
        out_shape=jax.ShapeDtypeStruct((VOCAB, DOUT), jnp.float32),
    )(table, W, b2d)


def _gather_kernel(n_total):
    n_rows = n_total // 128
    rows_per_w = n_rows // NW
    NBUF = 4
    mesh = plsc.VectorSubcoreMesh(core_axis_name="c", subcore_axis_name="s")

    @functools.partial(
        pl.kernel,
        mesh=mesh,
        out_type=jax.ShapeDtypeStruct((n_total, DOUT), jnp.float32),
        scratch_types=[
            pltpu.VMEM((rows_per_w, 128), jnp.int32),
            pltpu.VMEM((NBUF, 128, DOUT), jnp.float32),
            pltpu.SemaphoreType.DMA,
            pltpu.SemaphoreType.DMA,
        ],
    )
    def k(proj_hbm, idx_hbm, out_hbm, idx_v, rows, gsem, ssem):
        wid = lax.axis_index("s") * NC + lax.axis_index("c")
        row_base = wid * rows_per_w

        pltpu.sync_copy(idx_hbm.at[pl.ds(row_base, rows_per_w)], idx_v)

        def remap_row(j):
            for kk in range(128 // LANES):
                v = idx_v[j, pl.ds(kk * LANES, LANES)]
                v = jnp.where(v == -1, VOCAB - 2, v)
                v = jnp.where(v == -2, VOCAB - 1, v)
                v = jnp.clip(v, 0, VOCAB - 1)
                idx_v[j, pl.ds(kk * LANES, LANES)] = v

        def gather(g, b):
            return pltpu.make_async_copy(
                proj_hbm.at[idx_v.at[g]], rows.at[b], gsem
            )

        def store(g, b):
            return pltpu.make_async_copy(
                rows.at[b], out_hbm.at[pl.ds((row_base + g) * 128, 128)], ssem
            )

        remap_row(0)
        remap_row(1)
        gather(0, 0).start()
        gather(1, 1).start()

        def outer(o, carry):
            for b in range(NBUF):
                g = o * NBUF + b

                @pl.when(g + 2 < rows_per_w)
                def _remap_next(g=g):
                    remap_row(g + 2)

                gather(g, b).wait()

                @pl.when(g >= 2)
                def _wait_prev_store(b=b, g=g):
                    store(g, (b + 2) % NBUF).wait()

                @pl.when(g + 2 < rows_per_w)
                def _prefetch(b=b, g=g):
                    gather(g + 2, (b + 2) % NBUF).start()

                store(g, b).start()
            return carry

        lax.fori_loop(0, rows_per_w // NBUF, outer, 0)

        store(rows_per_w - 2, (rows_per_w - 2) % NBUF).wait()
        store(rows_per_w - 1, (rows_per_w - 1) % NBUF).wait()

    return k


def kernel(x, table, W, b):
    B, L = x.shape
    n_total = B * L
    proj = _project_table(table, W, b.reshape(1, DOUT))
    idx = x.reshape(n_total // 128, 128).astype(jnp.int32)
    out = _gather_kernel(n_total)(proj, idx)
    return out.reshape(B, L, DOUT)


# speedup vs baseline: 1.0128x; 1.0128x over previous
import functools

import jax
import jax.numpy as jnp
from jax import lax
from jax.experimental import pallas as pl
from jax.experimental.pallas import tpu as pltpu
from jax.experimental.pallas import tpu_sc as plsc

VOCAB = 100000
EMBED = 64
DOUT = 128

NC = 2
NS = 16
NW = NC * NS
LANES = 16


def _project_table(table, W, b2d):
    BLK = 10000

    def body(t_ref, w_ref, b_ref, o_ref):
        o_ref[...] = (
            jnp.dot(t_ref[...], w_ref[...], preferred_element_type=jnp.float32)
            + b_ref[...]
        )

    return pl.pallas_call(
        body,
        grid=(VOCAB // BLK,),
        in_specs=[
            pl.BlockSpec((BLK, EMBED), lambda i: (i, 0)),
            pl.BlockSpec((EMBED, DOUT), lambda i: (0, 0)),
            pl.BlockSpec((1, DOUT), lambda i: (0, 0)),
        ],
        out_specs=pl.BlockSpec((BLK, DOUT), lambda i: (i, 0)),
        out_shape=jax.ShapeDtypeStruct((VOCAB, DOUT), jnp.float32),
    )(table, W, b2d)


def _gather_kernel(n_total):
    n_rows = n_total // 128
    rows_per_w = n_rows // NW
    A = 3
    NBUF = 5
    mesh = plsc.VectorSubcoreMesh(core_axis_name="c", subcore_axis_name="s")

    @functools.partial(
        pl.kernel,
        mesh=mesh,
        out_type=jax.ShapeDtypeStruct((n_total, DOUT), jnp.float32),
        scratch_types=[
            pltpu.VMEM((rows_per_w, 128), jnp.int32),
            pltpu.VMEM((NBUF, 128, DOUT), jnp.float32),
            pltpu.SemaphoreType.DMA,
            pltpu.SemaphoreType.DMA,
        ],
    )
    def k(proj_hbm, idx_hbm, out_hbm, idx_v, rows, gsem, ssem):
        wid = lax.axis_index("s") * NC + lax.axis_index("c")
        row_base = wid * rows_per_w

        pltpu.sync_copy(idx_hbm.at[pl.ds(row_base, rows_per_w)], idx_v)

        def remap_row(j):
            for kk in range(128 // LANES):
                v = idx_v[j, pl.ds(kk * LANES, LANES)]
                v = jnp.where(v == -1, VOCAB - 2, v)
                v = jnp.where(v == -2, VOCAB - 1, v)
                v = jnp.clip(v, 0, VOCAB - 1)
                idx_v[j, pl.ds(kk * LANES, LANES)] = v

        def gather(g, b):
            return pltpu.make_async_copy(
                proj_hbm.at[idx_v.at[g]], rows.at[b], gsem
            )

        def store(g, b):
            return pltpu.make_async_copy(
                rows.at[b], out_hbm.at[pl.ds((row_base + g) * 128, 128)], ssem
            )

        for g0 in range(A):
            remap_row(g0)
            gather(g0, g0).start()

        def outer(o, carry):
            for b in range(NBUF):
                g = o * NBUF + b

                @pl.when(g + A < rows_per_w)
                def _remap_next(g=g):
                    remap_row(g + A)

                gather(g, b).wait()

                @pl.when(g >= NBUF - A)
                def _wait_prev_store(b=b, g=g):
                    store(g, (b + A) % NBUF).wait()

                @pl.when(g + A < rows_per_w)
                def _prefetch(b=b, g=g):
                    gather(g + A, (b + A) % NBUF).start()

                store(g, b).start()
            return carry

        lax.fori_loop(0, rows_per_w // NBUF, outer, 0)

        for g0 in range(rows_per_w - (NBUF - A), rows_per_w):
            store(g0, g0 % NBUF).wait()

    return k


def kernel(x, table, W, b):
    B, L = x.shape
    n_total = B * L
    proj = _project_table(table, W, b.reshape(1, DOUT))
    idx = x.reshape(n_total // 128, 128).astype(jnp.int32)
    out = _gather_kernel(n_total)(proj, idx)
    return out.reshape(B, L, DOUT)
